# baseline (speedup 1.0000x reference)
import jax
import jax.numpy as jnp
from jax.experimental import pallas as pl
from jax.experimental.pallas import tpu as pltpu

_THW = 25


def _fused_kernel(x_ref, pw_ref, pb_ref, w1_ref, b1_ref, w2_ref, b2_ref,
                  o_ref, h_ref):
    i = pl.program_id(0)
    thw, b, c = x_ref.shape

    pw = pw_ref[...].astype(jnp.bfloat16)
    x = x_ref[...].astype(jnp.bfloat16).reshape(thw * b, c)
    p2 = jnp.dot(x, pw, preferred_element_type=jnp.float32)
    p2 = p2 + pb_ref[...]
    p3 = p2.reshape(thw, b, 8)

    d = None
    for ci in range(8):
        pc = jnp.swapaxes(p3[:, :, ci], 0, 1)
        pc = pc.astype(jnp.bfloat16)
        w1c = w1_ref[ci, 0].astype(jnp.bfloat16)
        dc = jnp.dot(pc, w1c, preferred_element_type=jnp.float32)
        d = dc if d is None else d + dc

    @pl.when(i == 0)
    def _():
        h_ref[...] = d

    @pl.when(i > 0)
    def _():
        h_ref[...] += d

    @pl.when(i == pl.num_programs(0) - 1)
    def _():
        hr = jnp.maximum(h_ref[...] + b1_ref[...], 0.0).astype(jnp.bfloat16)
        w2 = w2_ref[...].astype(jnp.bfloat16)
        y = jnp.dot(hr, w2, preferred_element_type=jnp.float32)
        o_ref[...] = y + b2_ref[...]


@jax.jit
def _forward(feat, pool_w, pool_b, fc1_w, fc1_b, fc2_w, fc2_b):
    B, C, H, W = feat.shape
    HW = H * W
    NT = HW // _THW
    N1 = fc1_w.shape[1]
    N2 = fc2_w.shape[1]

    xt = feat.reshape(B, C, HW).transpose(2, 0, 1)
    w1v = fc1_w.reshape(8, NT, _THW, N1)

    y = pl.pallas_call(
        _fused_kernel,
        out_shape=jax.ShapeDtypeStruct((B, N2), jnp.float32),
        grid=(NT,),
        in_specs=[
            pl.BlockSpec((_THW, B, C), lambda i: (i, 0, 0)),
            pl.BlockSpec((C, 8), lambda i: (0, 0)),
            pl.BlockSpec((1, 8), lambda i: (0, 0)),
            pl.BlockSpec((8, 1, _THW, N1), lambda i: (0, i, 0, 0)),
            pl.BlockSpec((1, N1), lambda i: (0, 0)),
            pl.BlockSpec((N1, N2), lambda i: (0, 0)),
            pl.BlockSpec((1, N2), lambda i: (0, 0)),
        ],
        out_specs=pl.BlockSpec((B, N2), lambda i: (0, 0)),
        scratch_shapes=[pltpu.VMEM((B, N1), jnp.float32)],
        compiler_params=pltpu.CompilerParams(
            dimension_semantics=("arbitrary",)),
    )(xt, pool_w.T, pool_b.reshape(1, 8), w1v, fc1_b.reshape(1, N1),
      fc2_w, fc2_b.reshape(1, N2))
    return y.reshape(B, 37, 10, 4)


def kernel(feat, pool_w, pool_b, fc1_w, fc1_b, fc2_w, fc2_b):
    return _forward(feat, pool_w, pool_b, fc1_w, fc1_b, fc2_w, fc2_b)


# speedup vs baseline: 1.5061x; 1.5061x over previous
import jax
import jax.numpy as jnp
from jax.experimental import pallas as pl
from jax.experimental.pallas import tpu as pltpu

_THW = 25
_NA = 9
_TN = 256
_NB = 8


def _fused_kernel(x_ref, pw_ref, pb_ref, w1_ref, b1_ref, w2_ref, b2_ref,
                  o_ref, p_ref, p2_ref):
    i = pl.program_id(0)
    thw, b, c = x_ref.shape

    @pl.when(i == 0)
    def _():
        o_ref[...] = jnp.broadcast_to(b2_ref[...], o_ref.shape)

    @pl.when(i < _NA)
    def _():
        pw = pw_ref[...].astype(jnp.bfloat16)
        x = x_ref[...].astype(jnp.bfloat16).reshape(thw * b, c)
        p2 = jnp.dot(x, pw, preferred_element_type=jnp.float32)
        p3 = (p2 + pb_ref[...]).reshape(thw, b, 8)
        p_ref[pl.ds(i * thw, thw), :, :] = p3

    @pl.when(i == _NA)
    def _():
        pall = p_ref[...]
        p2_ref[...] = pall.transpose(2, 0, 1).reshape(
            8 * pall.shape[0], b).astype(jnp.bfloat16)

    @pl.when(i >= _NA)
    def _():
        w1 = w1_ref[...].astype(jnp.bfloat16)
        hj = jax.lax.dot_general(
            p2_ref[...], w1, (((0,), (0,)), ((), ())),
            preferred_element_type=jnp.float32)
        hj = jnp.maximum(hj + b1_ref[...], 0.0).astype(jnp.bfloat16)
        w2 = w2_ref[...].astype(jnp.bfloat16)
        o_ref[...] += jnp.dot(hj, w2, preferred_element_type=jnp.float32)


@jax.jit
def _forward(feat, pool_w, pool_b, fc1_w, fc1_b, fc2_w, fc2_b):
    B, C, H, W = feat.shape
    HW = H * W
    K1 = fc1_w.shape[0]
    N1 = fc1_w.shape[1]
    N2 = fc2_w.shape[1]

    xt = feat.reshape(B, C, HW).transpose(2, 0, 1)

    y = pl.pallas_call(
        _fused_kernel,
        out_shape=jax.ShapeDtypeStruct((B, N2), jnp.float32),
        grid=(_NA + _NB,),
        in_specs=[
            pl.BlockSpec((_THW, B, C),
                         lambda i: (jnp.minimum(i, _NA - 1), 0, 0)),
            pl.BlockSpec((C, 8), lambda i: (0, 0)),
            pl.BlockSpec((1, 8), lambda i: (0, 0)),
            pl.BlockSpec((K1, _TN),
                         lambda i: (0, jnp.maximum(i - _NA, 0))),
            pl.BlockSpec((1, _TN),
                         lambda i: (0, jnp.maximum(i - _NA, 0))),
            pl.BlockSpec((_TN, N2),
                         lambda i: (jnp.maximum(i - _NA, 0), 0)),
            pl.BlockSpec((1, N2), lambda i: (0, 0)),
        ],
        out_specs=pl.BlockSpec((B, N2), lambda i: (0, 0)),
        scratch_shapes=[pltpu.VMEM((HW, B, 8), jnp.float32),
                        pltpu.VMEM((K1, B), jnp.bfloat16)],
        compiler_params=pltpu.CompilerParams(
            dimension_semantics=("arbitrary",)),
    )(xt, pool_w.T, pool_b.reshape(1, 8), fc1_w, fc1_b.reshape(1, N1),
      fc2_w, fc2_b.reshape(1, N2))
    return y.reshape(B, 37, 10, 4)


def kernel(feat, pool_w, pool_b, fc1_w, fc1_b, fc2_w, fc2_b):
    return _forward(feat, pool_w, pool_b, fc1_w, fc1_b, fc2_w, fc2_b)


# speedup vs baseline: 1.5931x; 1.0578x over previous
import jax
import jax.numpy as jnp
from jax.experimental import pallas as pl
from jax.experimental.pallas import tpu as pltpu

_THW = 45
_NA = 5
_TN = 512
_NB = 4


def _fused_kernel(x_ref, pw_ref, pb_ref, w1_ref, b1_ref, w2_ref, b2_ref,
                  o_ref, p_ref, p2_ref):
    i = pl.program_id(0)
    thw, b, c = x_ref.shape

    @pl.when(i == 0)
    def _():
        o_ref[...] = jnp.broadcast_to(b2_ref[...], o_ref.shape)

    @pl.when(i < _NA)
    def _():
        pw = pw_ref[...].astype(jnp.bfloat16)
        x = x_ref[...].astype(jnp.bfloat16).reshape(thw * b, c)
        p2 = jnp.dot(x, pw, preferred_element_type=jnp.float32)
        p3 = (p2 + pb_ref[...]).reshape(thw, b, 8)
        p_ref[pl.ds(i * thw, thw), :, :] = p3

    @pl.when(i == _NA)
    def _():
        pall = p_ref[...]
        p2_ref[...] = pall.transpose(2, 0, 1).reshape(
            8 * pall.shape[0], b).astype(jnp.bfloat16)

    @pl.when(i >= _NA)
    def _():
        w1 = w1_ref[...].astype(jnp.bfloat16)
        hj = jax.lax.dot_general(
            p2_ref[...], w1, (((0,), (0,)), ((), ())),
            preferred_element_type=jnp.float32)
        hj = jnp.maximum(hj + b1_ref[...], 0.0).astype(jnp.bfloat16)
        w2 = w2_ref[...].astype(jnp.bfloat16)
        o_ref[...] += jnp.dot(hj, w2, preferred_element_type=jnp.float32)


@jax.jit
def _forward(feat, pool_w, pool_b, fc1_w, fc1_b, fc2_w, fc2_b):
    B, C, H, W = feat.shape
    HW = H * W
    K1 = fc1_w.shape[0]
    N1 = fc1_w.shape[1]
    N2 = fc2_w.shape[1]

    xt = feat.reshape(B, C, HW).transpose(2, 0, 1)

    y = pl.pallas_call(
        _fused_kernel,
        out_shape=jax.ShapeDtypeStruct((B, N2), jnp.float32),
        grid=(_NA + _NB,),
        in_specs=[
            pl.BlockSpec((_THW, B, C),
                         lambda i: (jnp.minimum(i, _NA - 1), 0, 0)),
            pl.BlockSpec((C, 8), lambda i: (0, 0)),
            pl.BlockSpec((1, 8), lambda i: (0, 0)),
            pl.BlockSpec((K1, _TN),
                         lambda i: (0, jnp.maximum(i - _NA, 0))),
            pl.BlockSpec((1, _TN),
                         lambda i: (0, jnp.maximum(i - _NA, 0))),
            pl.BlockSpec((_TN, N2),
                         lambda i: (jnp.maximum(i - _NA, 0), 0)),
            pl.BlockSpec((1, N2), lambda i: (0, 0)),
        ],
        out_specs=pl.BlockSpec((B, N2), lambda i: (0, 0)),
        scratch_shapes=[pltpu.VMEM((HW, B, 8), jnp.float32),
                        pltpu.VMEM((K1, B), jnp.bfloat16)],
        compiler_params=pltpu.CompilerParams(
            dimension_semantics=("arbitrary",)),
    )(xt, pool_w.T, pool_b.reshape(1, 8), fc1_w, fc1_b.reshape(1, N1),
      fc2_w, fc2_b.reshape(1, N2))
    return y.reshape(B, 37, 10, 4)


def kernel(feat, pool_w, pool_b, fc1_w, fc1_b, fc2_w, fc2_b):
    return _forward(feat, pool_w, pool_b, fc1_w, fc1_b, fc2_w, fc2_b)


# speedup vs baseline: 2.2374x; 1.4044x over previous
import jax
import jax.numpy as jnp
from jax.experimental import pallas as pl
from jax.experimental.pallas import tpu as pltpu

_THW = 45
_NA = 5
_TN = 512
_NB = 4


def _fused_kernel(x_ref, pw_ref, pb_ref, w1_ref, b1_ref, w2t_ref, b2_ref,
                  o_ref, p_ref, p2_ref):
    i = pl.program_id(0)
    thw, b, c = x_ref.shape

    @pl.when(i == 0)
    def _():
        b2col = b2_ref[...].reshape(o_ref.shape[0], 1)
        o_ref[...] = jnp.broadcast_to(b2col, o_ref.shape)

    @pl.when(i < _NA)
    def _():
        pw = pw_ref[...].astype(jnp.bfloat16)
        x = x_ref[...].astype(jnp.bfloat16).reshape(thw * b, c)
        p2 = jax.lax.dot_general(x, pw, (((1,), (1,)), ((), ())),
                                 preferred_element_type=jnp.float32)
        p3 = (p2 + pb_ref[...]).reshape(thw, b, 8)
        p_ref[pl.ds(i * thw, thw), :, :] = p3

    @pl.when(i == _NA)
    def _():
        pall = p_ref[...]
        p2_ref[...] = pall.transpose(2, 0, 1).reshape(
            8 * pall.shape[0], b).astype(jnp.bfloat16)

    @pl.when(i >= _NA)
    def _():
        j = i - _NA
        w1 = w1_ref[...].astype(jnp.bfloat16)
        hj = jax.lax.dot_general(
            p2_ref[...], w1, (((0,), (0,)), ((), ())),
            preferred_element_type=jnp.float32)
        hj = jnp.maximum(hj + b1_ref[...], 0.0).astype(jnp.bfloat16)
        w2t = w2t_ref[:, pl.ds(j * _TN, _TN)].astype(jnp.bfloat16)
        o_ref[...] += jax.lax.dot_general(
            w2t, hj, (((1,), (1,)), ((), ())),
            preferred_element_type=jnp.float32)


@jax.jit
def _forward(feat, pool_w, pool_b, fc1_w, fc1_b, fc2_w, fc2_b):
    B, C, H, W = feat.shape
    HW = H * W
    K1 = fc1_w.shape[0]
    N1 = fc1_w.shape[1]
    N2 = fc2_w.shape[1]

    xt = feat.reshape(B, C, HW).transpose(2, 0, 1)
    w2t = fc2_w.T

    yt = pl.pallas_call(
        _fused_kernel,
        out_shape=jax.ShapeDtypeStruct((N2, B), jnp.float32),
        grid=(_NA + _NB,),
        in_specs=[
            pl.BlockSpec((_THW, B, C),
                         lambda i: (jnp.minimum(i, _NA - 1), 0, 0)),
            pl.BlockSpec((8, C), lambda i: (0, 0)),
            pl.BlockSpec((1, 8), lambda i: (0, 0)),
            pl.BlockSpec((K1, _TN),
                         lambda i: (0, jnp.maximum(i - _NA, 0))),
            pl.BlockSpec((1, _TN),
                         lambda i: (0, jnp.maximum(i - _NA, 0))),
            pl.BlockSpec((N2, N1), lambda i: (0, 0)),
            pl.BlockSpec((1, N2), lambda i: (0, 0)),
        ],
        out_specs=pl.BlockSpec((N2, B), lambda i: (0, 0)),
        scratch_shapes=[pltpu.VMEM((HW, B, 8), jnp.float32),
                        pltpu.VMEM((K1, B), jnp.bfloat16)],
        compiler_params=pltpu.CompilerParams(
            dimension_semantics=("arbitrary",)),
    )(xt, pool_w, pool_b.reshape(1, 8), fc1_w, fc1_b.reshape(1, N1),
      w2t, fc2_b.reshape(1, N2))
    return yt.T.reshape(B, 37, 10, 4)


def kernel(feat, pool_w, pool_b, fc1_w, fc1_b, fc2_w, fc2_b):
    return _forward(feat, pool_w, pool_b, fc1_w, fc1_b, fc2_w, fc2_b)


# speedup vs baseline: 2.3172x; 1.0357x over previous
import jax
import jax.numpy as jnp
from jax.experimental import pallas as pl
from jax.experimental.pallas import tpu as pltpu

_THW = 45
_NA = 5
_TN = 512
_NB = 4


def _fused_kernel(x_ref, pw_ref, pb_ref, w1_ref, b1_ref, w2t_ref, b2_ref,
                  o_ref, p_ref, p2_ref):
    i = pl.program_id(0)
    thw, b, c = x_ref.shape

    @pl.when(i == 0)
    def _():
        b2col = b2_ref[...].reshape(o_ref.shape[0], 1)
        o_ref[...] = jnp.broadcast_to(b2col, o_ref.shape)

    @pl.when(i < _NA)
    def _():
        pw = pw_ref[...].astype(jnp.bfloat16)
        x = x_ref[...].astype(jnp.bfloat16).reshape(thw * b, c)
        p2 = jax.lax.dot_general(x, pw, (((1,), (1,)), ((), ())),
                                 preferred_element_type=jnp.float32)
        p3 = (p2 + pb_ref[...]).reshape(thw, b, 8)
        p_ref[pl.ds(i * thw, thw), :, :] = p3

    @pl.when(i == _NA)
    def _():
        pall = p_ref[...]
        p2_ref[...] = pall.transpose(2, 0, 1).reshape(
            8 * pall.shape[0], b).astype(jnp.bfloat16)

    @pl.when(i >= _NA)
    def _():
        j = i - _NA
        w1 = w1_ref[...].astype(jnp.bfloat16)
        hj = jax.lax.dot_general(
            p2_ref[...], w1, (((0,), (0,)), ((), ())),
            preferred_element_type=jnp.float32)
        hj = jnp.maximum(hj + b1_ref[...], 0.0).astype(jnp.bfloat16)
        w2t = w2t_ref[:, pl.ds(j * _TN, _TN)].astype(jnp.bfloat16)
        o_ref[...] += jax.lax.dot_general(
            w2t, hj, (((1,), (1,)), ((), ())),
            preferred_element_type=jnp.float32)


@jax.jit
def _forward(feat, pool_w, pool_b, fc1_w, fc1_b, fc2_w, fc2_b):
    B, C, H, W = feat.shape
    HW = H * W
    K1 = fc1_w.shape[0]
    N1 = fc1_w.shape[1]
    N2 = fc2_w.shape[1]

    xt = feat.reshape(B, C, HW).transpose(2, 0, 1)
    w2t = fc2_w.T

    yt = pl.pallas_call(
        _fused_kernel,
        out_shape=jax.ShapeDtypeStruct((N2, B), jnp.float32),
        grid=(_NA + _NB,),
        in_specs=[
            pl.BlockSpec((_THW, B, C),
                         lambda i: (jnp.minimum(i, _NA - 1), 0, 0)),
            pl.BlockSpec((8, C), lambda i: (0, 0)),
            pl.BlockSpec((1, 8), lambda i: (0, 0)),
            pl.BlockSpec((K1, _TN),
                         lambda i: (0, jnp.maximum(i - _NA, 0))),
            pl.BlockSpec((1, _TN),
                         lambda i: (0, jnp.maximum(i - _NA, 0))),
            pl.BlockSpec((N2, N1), lambda i: (0, 0)),
            pl.BlockSpec((N2,), lambda i: (0,)),
        ],
        out_specs=pl.BlockSpec((N2, B), lambda i: (0, 0)),
        scratch_shapes=[pltpu.VMEM((HW, B, 8), jnp.float32),
                        pltpu.VMEM((K1, B), jnp.bfloat16)],
        compiler_params=pltpu.CompilerParams(
            dimension_semantics=("arbitrary",)),
    )(xt, pool_w, pool_b.reshape(1, 8), fc1_w, fc1_b.reshape(1, N1),
      w2t, fc2_b)
    return yt.T.reshape(B, 37, 10, 4)


def kernel(feat, pool_w, pool_b, fc1_w, fc1_b, fc2_w, fc2_b):
    return _forward(feat, pool_w, pool_b, fc1_w, fc1_b, fc2_w, fc2_b)
